# Initial kernel scaffold; baseline (speedup 1.0000x reference)
import jax
import jax.numpy as jnp
from jax.experimental import pallas as pl


def kernel(x, target):
    raise NotImplementedError("write your pallas kernel here")


# speedup vs baseline: 6.5548x; 6.5548x over previous
import functools
import math

import jax
import jax.numpy as jnp
from jax.experimental import pallas as pl
from jax.experimental.pallas import tpu as pltpu

_V = 32000
_SMOOTHING = 0.1
_CONF = 1.0 - _SMOOTHING
_EPS = _SMOOTHING / _V

_ROWS_PER_BLOCK = 64


def _loss_block_kernel(x_ref, tgt_ref, out_ref):
    x = x_ref[...]
    tgt = tgt_ref[0, 0, :]
    r, v = x.shape
    cols = jax.lax.broadcasted_iota(jnp.int32, (r, v), 1)
    hit = cols == tgt[:, None]
    gathered = jnp.sum(jnp.where(hit, x, 0.0))
    total = jnp.sum(x)
    partial = -_EPS * total - (_CONF - _EPS) * gathered
    out_ref[...] = partial.reshape(1, 1, 1)


@functools.partial(jax.jit, static_argnames=())
def kernel(x, target):
    n, v = x.shape
    r = _ROWS_PER_BLOCK
    g = n // r
    tgt3 = target.astype(jnp.int32).reshape(g, 1, r)
    partials = pl.pallas_call(
        _loss_block_kernel,
        grid=(g,),
        in_specs=[
            pl.BlockSpec((r, v), lambda i: (i, 0)),
            pl.BlockSpec((1, 1, r), lambda i: (i, 0, 0)),
        ],
        out_specs=pl.BlockSpec((1, 1, 1), lambda i: (i, 0, 0)),
        out_shape=jax.ShapeDtypeStruct((g, 1, 1), jnp.float32),
        compiler_params=pltpu.CompilerParams(
            dimension_semantics=("parallel",),
        ),
    )(x, tgt3)
    const = n * ((v - 1) * _EPS * math.log(_EPS) + _CONF * math.log(_CONF))
    return jnp.float32(const) + jnp.sum(partials)
